# baseline (speedup 1.0000x reference)
import functools

import jax
import jax.numpy as jnp
from jax import lax
from jax.experimental import pallas as pl
from jax.experimental.pallas import tpu as pltpu
from jax.experimental.pallas import tpu_sc as plsc

_NUM_CLASSES = 1000000
_HIDDEN = 32
_BATCH = 16384

_NC = 2
_NS = 16
_L = 16
_NW = _NC * _NS
_B_PER_W = _BATCH // _NW
_CHUNK = 128
_NCHUNK = _B_PER_W // _CHUNK

_mesh = plsc.VectorSubcoreMesh(core_axis_name="c", subcore_axis_name="s")


@functools.partial(
    pl.kernel,
    mesh=_mesh,
    compiler_params=pltpu.CompilerParams(use_tc_tiling_on_sc=False),
    out_type=jax.ShapeDtypeStruct((_BATCH, _HIDDEN), jnp.float32),
    scratch_types=[
        pltpu.VMEM((_NCHUNK, _CHUNK), jnp.int32),
        pltpu.VMEM((_NCHUNK, _CHUNK), jnp.int32),
        pltpu.VMEM((_NCHUNK, _CHUNK), jnp.int32),
        pltpu.VMEM((_B_PER_W, _HIDDEN), jnp.float32),
        pltpu.SemaphoreType.DMA,
    ],
)
def _embed(labels_hbm, drop_hbm, table_hbm, out_hbm,
           lab_v, drop_v, idx_v, rows_v, sem):
    wid = lax.axis_index("s") * _NC + lax.axis_index("c")
    base = wid * _B_PER_W

    pltpu.sync_copy(labels_hbm.at[wid], lab_v)
    pltpu.sync_copy(drop_hbm.at[wid], drop_v)

    for j in range(_NCHUNK):
        for i in range(_CHUNK // _L):
            sl = pl.ds(i * _L, _L)
            lab = lab_v[j, sl]
            dr = drop_v[j, sl]
            idx_v[j, sl] = jnp.where(dr != 0, jnp.full((_L,), _NUM_CLASSES,
                                                       jnp.int32), lab)

    copies = []
    for j in range(_NCHUNK):
        copies.append(pltpu.async_copy(
            table_hbm.at[idx_v.at[j]],
            rows_v.at[pl.ds(j * _CHUNK, _CHUNK)],
            sem,
        ))
    for c in copies:
        c.wait()

    pltpu.sync_copy(rows_v, out_hbm.at[pl.ds(base, _B_PER_W)])


def kernel(labels, force_drop_ids, table):
    lab = labels.astype(jnp.int32).reshape(_NW, _NCHUNK, _CHUNK)
    drop = force_drop_ids.astype(jnp.int32).reshape(_NW, _NCHUNK, _CHUNK)
    return _embed(lab, drop, table)
